# baseline (speedup 1.0000x reference)
import functools

import jax
import jax.numpy as jnp
from jax import lax
from jax.experimental import pallas as pl
from jax.experimental.pallas import tpu as pltpu
from jax.experimental.pallas import tpu_sc as plsc

ANCHORS_NUM = 9
NMS_OUT = 300
IOU_THR = 0.7

N_PIX = 2304
C_IN = 512
N_BOX = N_PIX * ANCHORS_NUM
ROWS = 162
LANES = 128
NEG_INF = float("-inf")

NW = 16
PER_W = N_BOX // NW
CH = PER_W // 16
KEEP_PAD = 304
BIG = 2**30


def _head_body(flat_ref, wc_ref, bc_ref, wb_ref, bb_ref, prob_ref, bbox_ref):
    flat = flat_ref[...]
    cls = jnp.dot(flat, wc_ref[...], preferred_element_type=jnp.float32) + bc_ref[...]
    left = jnp.concatenate([cls[:, 1:], cls[:, :1]], axis=1)
    right = jnp.concatenate([cls[:, -1:], cls[:, :-1]], axis=1)
    col = jax.lax.broadcasted_iota(jnp.int32, cls.shape, 1)
    partner = jnp.where(col % 2 == 0, left, right)
    m = jnp.maximum(cls, partner)
    e = jnp.exp(cls - m)
    ep = jnp.exp(partner - m)
    prob_ref[...] = e / (e + ep)
    bbox_ref[...] = (
        jnp.dot(flat, wb_ref[...], preferred_element_type=jnp.float32) + bb_ref[...]
    )


def _decode_body(anc_ref, t_ref, img_ref, x0_ref, y0_ref, x1_ref, y1_ref):
    a0 = anc_ref[0]
    a1 = anc_ref[1]
    a2 = anc_ref[2]
    a3 = anc_ref[3]
    t0 = t_ref[0]
    t1 = t_ref[1]
    t3 = t_ref[2]
    w = a3 - a1 + 1.0
    h = a2 - a0 + 1.0
    x = a0 + 0.5 * h
    y = a1 + 0.5 * w
    x_pred = t0 * h + x
    y_pred = t1 * w + y
    h_pred = jnp.exp(t3) * h
    x0 = x_pred - 0.5 * h_pred
    x1 = x_pred + 0.5 * h_pred
    y0 = y_pred - 0.5 * y_pred
    y1 = y_pred + 0.5 * y_pred
    x0_ref[...] = jnp.maximum(x0, 0.0)
    x1_ref[...] = jnp.minimum(x1, img_ref[0])
    y0_ref[...] = jnp.maximum(y0, 0.0)
    y1_ref[...] = jnp.minimum(y1, img_ref[1])


def _bfly_max(v, li):
    for sh in (1, 2, 4, 8):
        v = jnp.maximum(v, v.at[li ^ sh].get(mode="promise_in_bounds"))
    return v


def _bfly_min(v, li):
    for sh in (1, 2, 4, 8):
        v = jnp.minimum(v, v.at[li ^ sh].get(mode="promise_in_bounds"))
    return v


def _nms_sc_body(planes_h, out_h,
                 x0_v, y0_v, x1_v, y1_v, s_v,
                 stage_v, merge_v, keep_v, pub_sh):
    wid = lax.axis_index("s")
    base = wid * PER_W
    li = lax.iota(jnp.int32, 16)
    minf = jnp.full((16,), NEG_INF, jnp.float32)
    bigv = jnp.full((16,), BIG, jnp.int32)

    pltpu.sync_copy(planes_h.at[pl.ds(0 * N_BOX + base, PER_W)], x0_v)
    pltpu.sync_copy(planes_h.at[pl.ds(1 * N_BOX + base, PER_W)], y0_v)
    pltpu.sync_copy(planes_h.at[pl.ds(2 * N_BOX + base, PER_W)], x1_v)
    pltpu.sync_copy(planes_h.at[pl.ds(3 * N_BOX + base, PER_W)], y1_v)
    pltpu.sync_copy(planes_h.at[pl.ds(4 * N_BOX + base, PER_W)], s_v)

    @pl.when(wid == 0)
    def _():
        keep_v[pl.ds(KEEP_PAD - 16, 16)] = jnp.full((16,), -1, jnp.int32)

    def prime(j, carry):
        macc, iacc = carry
        sl = pl.ds(j * 16, 16)
        sc = s_v[sl]
        lin = base + j * 16 + li
        upd = sc > macc
        return jnp.where(upd, sc, macc), jnp.where(upd, lin, iacc)

    macc0, iacc0 = lax.fori_loop(0, CH, prime, (minf, bigv))

    def iter_body(i, carry):
        macc, iacc = carry
        m_loc = _bfly_max(macc, li)
        i_loc = _bfly_min(jnp.where(macc == m_loc, iacc, BIG), li)
        pv = jnp.clip(i_loc - base, 0, PER_W - 1)
        gx0 = plsc.load_gather(x0_v, [pv])
        gy0 = plsc.load_gather(y0_v, [pv])
        gx1 = plsc.load_gather(x1_v, [pv])
        gy1 = plsc.load_gather(y1_v, [pv])
        idxf = plsc.bitcast(i_loc, jnp.float32)
        v = jnp.where(li == 1, idxf, m_loc)
        v = jnp.where(li == 2, gx0, v)
        v = jnp.where(li == 3, gy0, v)
        v = jnp.where(li == 4, gx1, v)
        v = jnp.where(li == 5, gy1, v)
        stage_v[...] = v
        par = i % 2
        pltpu.sync_copy(stage_v, pub_sh.at[par, 0, wid])
        pltpu.sync_copy(stage_v, pub_sh.at[par, 1, wid])
        plsc.subcore_barrier()
        pltpu.sync_copy(pub_sh.at[par, 1], merge_v)

        zero16 = jnp.zeros((16,), jnp.int32)
        scores = plsc.load_gather(merge_v, [li, zero16])
        idxcol = plsc.bitcast(
            plsc.load_gather(merge_v, [li, zero16 + 1]), jnp.int32)
        m_g = _bfly_max(scores, li)
        tie = scores == m_g
        idx_g = _bfly_min(jnp.where(tie, idxcol, BIG), li)
        rv = _bfly_min(jnp.where(tie & (idxcol == idx_g), li, BIG), li)
        bx0 = plsc.load_gather(merge_v, [rv, zero16 + 2])
        by0 = plsc.load_gather(merge_v, [rv, zero16 + 3])
        bx1 = plsc.load_gather(merge_v, [rv, zero16 + 4])
        by1 = plsc.load_gather(merge_v, [rv, zero16 + 5])
        validv = m_g > NEG_INF

        @pl.when(wid == 0)
        def _():
            kvec = jnp.where(validv, idx_g, -1)
            plsc.store_scatter(keep_v, [jnp.broadcast_to(i, (16,))], kvec,
                               mask=li == 0)

        barea = (jnp.maximum(bx1 - bx0, 0.0) * jnp.maximum(by1 - by0, 0.0))

        def supp(j, carry2):
            macc2, iacc2 = carry2
            sl = pl.ds(j * 16, 16)
            x0c = x0_v[sl]
            y0c = y0_v[sl]
            x1c = x1_v[sl]
            y1c = y1_v[sl]
            arc = (jnp.maximum(x1c - x0c, 0.0) * jnp.maximum(y1c - y0c, 0.0))
            sc = s_v[sl]
            linc = base + j * 16 + li
            yy1 = jnp.maximum(bx0, x0c)
            xx1 = jnp.maximum(by0, y0c)
            yy2 = jnp.minimum(bx1, x1c)
            xx2 = jnp.minimum(by1, y1c)
            inter = (jnp.maximum(yy2 - yy1, 0.0) * jnp.maximum(xx2 - xx1, 0.0))
            union = (barea + arc) - inter
            iou = jnp.where(union > 0.0, inter / union, 0.0)
            kill = ((iou > IOU_THR) | (linc == idx_g)) & validv
            s_new = jnp.where(kill, NEG_INF, sc)
            s_v[sl] = s_new
            upd = s_new > macc2
            return jnp.where(upd, s_new, macc2), jnp.where(upd, linc, iacc2)

        return lax.fori_loop(0, CH, supp, (minf, bigv))

    lax.fori_loop(0, NMS_OUT, iter_body, (macc0, iacc0))

    @pl.when(wid == 0)
    def _():
        pltpu.sync_copy(keep_v, out_h)
        pltpu.sync_copy(keep_v, out_h)


@functools.cache
def _nms_sc():
    return pl.kernel(
        _nms_sc_body,
        out_type=jax.ShapeDtypeStruct((KEEP_PAD,), jnp.int32),
        mesh=plsc.VectorSubcoreMesh(
            core_axis_name="c", subcore_axis_name="s",
            num_cores=1, num_subcores=NW),
        compiler_params=pltpu.CompilerParams(needs_layout_passes=False),
        scratch_types=[
            pltpu.VMEM((PER_W,), jnp.float32),
            pltpu.VMEM((PER_W,), jnp.float32),
            pltpu.VMEM((PER_W,), jnp.float32),
            pltpu.VMEM((PER_W,), jnp.float32),
            pltpu.VMEM((PER_W,), jnp.float32),
            pltpu.VMEM((16,), jnp.float32),
            pltpu.VMEM((NW, 16), jnp.float32),
            pltpu.VMEM((KEEP_PAD,), jnp.int32),
            pltpu.VMEM_SHARED((2, 2, NW, 16), jnp.float32),
        ],
    )


@jax.jit
def kernel(rpn_feature, anchors, img_sz, W_cls, b_cls, W_bbox, b_bbox):
    flat = rpn_feature.reshape(N_PIX, C_IN)
    prob, bbox = pl.pallas_call(
        _head_body,
        out_shape=(
            jax.ShapeDtypeStruct((N_PIX, 2 * ANCHORS_NUM), jnp.float32),
            jax.ShapeDtypeStruct((N_PIX, 4 * ANCHORS_NUM), jnp.float32),
        ),
    )(flat, W_cls, b_cls.reshape(1, -1), W_bbox, b_bbox.reshape(1, -1))

    s_in = prob[:, ANCHORS_NUM:].reshape(-1)
    anc = anchors.T.reshape(4, ROWS, LANES)
    pred_t = bbox.reshape(N_BOX, 4).T
    tsel = jnp.concatenate([pred_t[0:2], pred_t[3:4]], axis=0)
    tpl = tsel.reshape(3, ROWS, LANES)

    x0, y0, x1, y1 = pl.pallas_call(
        _decode_body,
        out_shape=(jax.ShapeDtypeStruct((ROWS, LANES), jnp.float32),) * 4,
        in_specs=[
            pl.BlockSpec(),
            pl.BlockSpec(),
            pl.BlockSpec(memory_space=pltpu.SMEM),
        ],
    )(anc, tpl, img_sz)

    planes = jnp.stack([x0.reshape(-1), y0.reshape(-1), x1.reshape(-1),
                        y1.reshape(-1), s_in]).reshape(-1)

    keep = _nms_sc()(planes)
    return keep[:NMS_OUT]


# speedup vs baseline: 1.1203x; 1.1203x over previous
import functools

import jax
import jax.numpy as jnp
from jax import lax
from jax.experimental import pallas as pl
from jax.experimental.pallas import tpu as pltpu
from jax.experimental.pallas import tpu_sc as plsc

ANCHORS_NUM = 9
NMS_OUT = 300
IOU_THR = 0.7

N_PIX = 2304
C_IN = 512
N_BOX = N_PIX * ANCHORS_NUM
ROWS = 162
LANES = 128
NEG_INF = float("-inf")

NW = 16
PER_W = N_BOX // NW
CH = PER_W // 16
KEEP_PAD = 304
BIG = 2**30


def _head_body(flat_ref, wc_ref, bc_ref, wb_ref, bb_ref, prob_ref, bbox_ref):
    flat = flat_ref[...]
    cls = jnp.dot(flat, wc_ref[...], preferred_element_type=jnp.float32) + bc_ref[...]
    left = jnp.concatenate([cls[:, 1:], cls[:, :1]], axis=1)
    right = jnp.concatenate([cls[:, -1:], cls[:, :-1]], axis=1)
    col = jax.lax.broadcasted_iota(jnp.int32, cls.shape, 1)
    partner = jnp.where(col % 2 == 0, left, right)
    m = jnp.maximum(cls, partner)
    e = jnp.exp(cls - m)
    ep = jnp.exp(partner - m)
    prob_ref[...] = e / (e + ep)
    bbox_ref[...] = (
        jnp.dot(flat, wb_ref[...], preferred_element_type=jnp.float32) + bb_ref[...]
    )


def _decode_body(anc_ref, t_ref, img_ref, x0_ref, y0_ref, x1_ref, y1_ref):
    a0 = anc_ref[0]
    a1 = anc_ref[1]
    a2 = anc_ref[2]
    a3 = anc_ref[3]
    t0 = t_ref[0]
    t1 = t_ref[1]
    t3 = t_ref[2]
    w = a3 - a1 + 1.0
    h = a2 - a0 + 1.0
    x = a0 + 0.5 * h
    y = a1 + 0.5 * w
    x_pred = t0 * h + x
    y_pred = t1 * w + y
    h_pred = jnp.exp(t3) * h
    x0 = x_pred - 0.5 * h_pred
    x1 = x_pred + 0.5 * h_pred
    y0 = y_pred - 0.5 * y_pred
    y1 = y_pred + 0.5 * y_pred
    x0_ref[...] = jnp.maximum(x0, 0.0)
    x1_ref[...] = jnp.minimum(x1, img_ref[0])
    y0_ref[...] = jnp.maximum(y0, 0.0)
    y1_ref[...] = jnp.minimum(y1, img_ref[1])


def _bfly_max(v, li):
    for sh in (1, 2, 4, 8):
        v = jnp.maximum(v, v.at[li ^ sh].get(mode="promise_in_bounds"))
    return v


def _bfly_min(v, li):
    for sh in (1, 2, 4, 8):
        v = jnp.minimum(v, v.at[li ^ sh].get(mode="promise_in_bounds"))
    return v


def _nms_sc_body(planes_h, out_h,
                 x0_v, y0_v, x1_v, y1_v, ar_v, s_v,
                 stage_v, merge_v, keep_v, pub_sh):
    wid = lax.axis_index("s")
    base = wid * PER_W
    li = lax.iota(jnp.int32, 16)
    minf = jnp.full((16,), NEG_INF, jnp.float32)
    bigv = jnp.full((16,), BIG, jnp.int32)

    pltpu.sync_copy(planes_h.at[pl.ds(0 * N_BOX + base, PER_W)], x0_v)
    pltpu.sync_copy(planes_h.at[pl.ds(1 * N_BOX + base, PER_W)], y0_v)
    pltpu.sync_copy(planes_h.at[pl.ds(2 * N_BOX + base, PER_W)], x1_v)
    pltpu.sync_copy(planes_h.at[pl.ds(3 * N_BOX + base, PER_W)], y1_v)
    pltpu.sync_copy(planes_h.at[pl.ds(4 * N_BOX + base, PER_W)], s_v)

    @pl.when(wid == 0)
    def _():
        keep_v[pl.ds(KEEP_PAD - 16, 16)] = jnp.full((16,), -1, jnp.int32)

    def prime(j, carry):
        macc, iacc = carry
        sl = pl.ds(j * 16, 16)
        x0c = x0_v[sl]
        y0c = y0_v[sl]
        x1c = x1_v[sl]
        y1c = y1_v[sl]
        ar_v[sl] = jnp.maximum(x1c - x0c, 0.0) * jnp.maximum(y1c - y0c, 0.0)
        sc = s_v[sl]
        lin = base + j * 16 + li
        upd = sc > macc
        return jnp.where(upd, sc, macc), jnp.where(upd, lin, iacc)

    macc0, iacc0 = lax.fori_loop(0, CH, prime, (minf, bigv))

    def iter_body(i, carry):
        macc, iacc = carry
        m_loc = _bfly_max(macc, li)
        i_loc = _bfly_min(jnp.where(macc == m_loc, iacc, BIG), li)
        pv = jnp.clip(i_loc - base, 0, PER_W - 1)
        gx0 = plsc.load_gather(x0_v, [pv])
        gy0 = plsc.load_gather(y0_v, [pv])
        gx1 = plsc.load_gather(x1_v, [pv])
        gy1 = plsc.load_gather(y1_v, [pv])
        idxf = plsc.bitcast(i_loc, jnp.float32)
        v = jnp.where(li == 1, idxf, m_loc)
        v = jnp.where(li == 2, gx0, v)
        v = jnp.where(li == 3, gy0, v)
        v = jnp.where(li == 4, gx1, v)
        v = jnp.where(li == 5, gy1, v)
        stage_v[...] = v
        par = i % 2
        pltpu.sync_copy(stage_v, pub_sh.at[par, 0, wid])
        pltpu.sync_copy(stage_v, pub_sh.at[par, 1, wid])
        plsc.subcore_barrier()
        pltpu.sync_copy(pub_sh.at[par, 1], merge_v)

        zero16 = jnp.zeros((16,), jnp.int32)
        scores = plsc.load_gather(merge_v, [li, zero16])
        m_g = _bfly_max(scores, li)
        rv = _bfly_min(jnp.where(scores == m_g, li, BIG), li)
        idx_g = plsc.bitcast(
            plsc.load_gather(merge_v, [rv, zero16 + 1]), jnp.int32)
        bx0 = plsc.load_gather(merge_v, [rv, zero16 + 2])
        by0 = plsc.load_gather(merge_v, [rv, zero16 + 3])
        bx1 = plsc.load_gather(merge_v, [rv, zero16 + 4])
        by1 = plsc.load_gather(merge_v, [rv, zero16 + 5])
        validv = m_g > NEG_INF

        @pl.when(wid == 0)
        def _():
            kvec = jnp.where(validv, idx_g, -1)
            plsc.store_scatter(keep_v, [jnp.broadcast_to(i, (16,))], kvec,
                               mask=li == 0)

        barea = (jnp.maximum(bx1 - bx0, 0.0) * jnp.maximum(by1 - by0, 0.0))

        def supp(j, carry2):
            macc2, iacc2 = carry2
            sl = pl.ds(j * 16, 16)
            x0c = x0_v[sl]
            y0c = y0_v[sl]
            x1c = x1_v[sl]
            y1c = y1_v[sl]
            arc = ar_v[sl]
            sc = s_v[sl]
            linc = base + j * 16 + li
            yy1 = jnp.maximum(bx0, x0c)
            xx1 = jnp.maximum(by0, y0c)
            yy2 = jnp.minimum(bx1, x1c)
            xx2 = jnp.minimum(by1, y1c)
            inter = (jnp.maximum(yy2 - yy1, 0.0) * jnp.maximum(xx2 - xx1, 0.0))
            union = (barea + arc) - inter
            iou = jnp.where(union > 0.0, inter / union, 0.0)
            kill = ((iou > IOU_THR) | (linc == idx_g)) & validv
            s_new = jnp.where(kill, NEG_INF, sc)
            s_v[sl] = s_new
            upd = s_new > macc2
            return jnp.where(upd, s_new, macc2), jnp.where(upd, linc, iacc2)

        return lax.fori_loop(0, CH, supp, (minf, bigv))

    lax.fori_loop(0, NMS_OUT, iter_body, (macc0, iacc0))

    @pl.when(wid == 0)
    def _():
        pltpu.sync_copy(keep_v, out_h)
        pltpu.sync_copy(keep_v, out_h)


@functools.cache
def _nms_sc():
    return pl.kernel(
        _nms_sc_body,
        out_type=jax.ShapeDtypeStruct((KEEP_PAD,), jnp.int32),
        mesh=plsc.VectorSubcoreMesh(
            core_axis_name="c", subcore_axis_name="s",
            num_cores=1, num_subcores=NW),
        compiler_params=pltpu.CompilerParams(needs_layout_passes=False),
        scratch_types=[
            pltpu.VMEM((PER_W,), jnp.float32),
            pltpu.VMEM((PER_W,), jnp.float32),
            pltpu.VMEM((PER_W,), jnp.float32),
            pltpu.VMEM((PER_W,), jnp.float32),
            pltpu.VMEM((PER_W,), jnp.float32),
            pltpu.VMEM((PER_W,), jnp.float32),
            pltpu.VMEM((16,), jnp.float32),
            pltpu.VMEM((NW, 16), jnp.float32),
            pltpu.VMEM((KEEP_PAD,), jnp.int32),
            pltpu.VMEM_SHARED((2, 2, NW, 16), jnp.float32),
        ],
    )


@jax.jit
def kernel(rpn_feature, anchors, img_sz, W_cls, b_cls, W_bbox, b_bbox):
    flat = rpn_feature.reshape(N_PIX, C_IN)
    prob, bbox = pl.pallas_call(
        _head_body,
        out_shape=(
            jax.ShapeDtypeStruct((N_PIX, 2 * ANCHORS_NUM), jnp.float32),
            jax.ShapeDtypeStruct((N_PIX, 4 * ANCHORS_NUM), jnp.float32),
        ),
    )(flat, W_cls, b_cls.reshape(1, -1), W_bbox, b_bbox.reshape(1, -1))

    s_in = prob[:, ANCHORS_NUM:].reshape(-1)
    anc = anchors.T.reshape(4, ROWS, LANES)
    pred_t = bbox.reshape(N_BOX, 4).T
    tsel = jnp.concatenate([pred_t[0:2], pred_t[3:4]], axis=0)
    tpl = tsel.reshape(3, ROWS, LANES)

    x0, y0, x1, y1 = pl.pallas_call(
        _decode_body,
        out_shape=(jax.ShapeDtypeStruct((ROWS, LANES), jnp.float32),) * 4,
        in_specs=[
            pl.BlockSpec(),
            pl.BlockSpec(),
            pl.BlockSpec(memory_space=pltpu.SMEM),
        ],
    )(anc, tpl, img_sz)

    planes = jnp.stack([x0.reshape(-1), y0.reshape(-1), x1.reshape(-1),
                        y1.reshape(-1), s_in]).reshape(-1)

    keep = _nms_sc()(planes)
    return keep[:NMS_OUT]
